# Initial kernel scaffold; baseline (speedup 1.0000x reference)
import jax
import jax.numpy as jnp
from jax.experimental import pallas as pl


def kernel(x, edge_index, edge_label_index, W1, b1, W2, b2):
    raise NotImplementedError("write your pallas kernel here")


# speedup vs baseline: 11.3005x; 11.3005x over previous
import functools

import jax
import jax.numpy as jnp
from jax import lax
from jax.experimental import pallas as pl
from jax.experimental.pallas import tpu as pltpu
from jax.experimental.pallas import tpu_sc as plsc

NC = 2
NS = 16
NW = NC * NS
LANES = 16

_MESH = plsc.VectorSubcoreMesh(core_axis_name="c", subcore_axis_name="s")
_SC_PARAMS = pltpu.CompilerParams(use_tc_tiling_on_sc=False,
                                  needs_layout_passes=False)


def _zero_vmem2d(ref, rows, cols):
    z = jnp.zeros((LANES,), jnp.float32)

    def zrow(r, carry):
        def zcol(j, c2):
            ref[r, pl.ds(j * LANES, LANES)] = z
            return c2
        return lax.fori_loop(0, cols // LANES, zcol, carry)

    lax.fori_loop(0, rows, zrow, 0)


def _zero_vmem1d(ref, n):
    z = jnp.zeros((LANES,), jnp.float32)

    def zi(i, c):
        ref[pl.ds(i * LANES, LANES)] = z
        return c

    lax.fori_loop(0, n // LANES, zi, 0)


def _make_deg_kernel(E, NP):
    CH = 80
    EW = E // NW
    n_ch = EW // CH
    span = NP // NS

    @functools.partial(
        pl.kernel,
        out_type=jax.ShapeDtypeStruct((NC * NP,), jnp.float32),
        mesh=_MESH,
        compiler_params=_SC_PARAMS,
        scratch_types=dict(
            deg_sh=pltpu.VMEM_SHARED((NP,), jnp.float32),
            idx_v=pltpu.VMEM((CH,), jnp.int32),
            ones_v=pltpu.VMEM((CH,), jnp.float32),
            zbuf=pltpu.VMEM((span,), jnp.float32),
        ),
    )
    def deg_kernel(dst_hbm, out_hbm, deg_sh, idx_v, ones_v, zbuf):
        c = lax.axis_index("c")
        s = lax.axis_index("s")
        one = jnp.ones((LANES,), jnp.float32)

        def fo(i, cc):
            ones_v[pl.ds(i * LANES, LANES)] = one
            return cc

        lax.fori_loop(0, CH // LANES, fo, 0)
        _zero_vmem1d(zbuf, span)
        pltpu.sync_copy(zbuf, deg_sh.at[pl.ds(s * span, span)])
        plsc.subcore_barrier()

        base = (c * NS + s) * EW

        def step(i, cc):
            pltpu.sync_copy(dst_hbm.at[pl.ds(base + i * CH, CH)], idx_v)
            pltpu.sync_copy(ones_v, deg_sh.at[idx_v], add=True)
            return cc

        lax.fori_loop(0, n_ch, step, 0)
        plsc.subcore_barrier()
        pltpu.sync_copy(deg_sh.at[pl.ds(s * span, span)],
                        out_hbm.at[pl.ds(c * NP + s * span, span)])

    return deg_kernel


def _make_scatter_kernel(NPAD, D, E):
    CH = 80
    EW = E // NW
    n_ch = EW // CH
    span = NPAD // NS
    zrows = 128

    @functools.partial(
        pl.kernel,
        out_type=jax.ShapeDtypeStruct((NC * NPAD, D), jnp.float32),
        mesh=_MESH,
        compiler_params=_SC_PARAMS,
        scratch_types=dict(
            acc_sh=pltpu.VMEM_SHARED((NPAD, D), jnp.float32),
            idx_s=pltpu.VMEM((CH,), jnp.int32),
            idx_d=pltpu.VMEM((CH,), jnp.int32),
            rows_v=pltpu.VMEM((CH, D), jnp.float32),
            zbuf=pltpu.VMEM((zrows, D), jnp.float32),
            sem=pltpu.SemaphoreType.DMA,
        ),
    )
    def scatter_kernel(u_hbm, src_hbm, dst_hbm, out_hbm,
                       acc_sh, idx_s, idx_d, rows_v, zbuf, sem):
        c = lax.axis_index("c")
        s = lax.axis_index("s")
        _zero_vmem2d(zbuf, zrows, D)
        for k in range(span // zrows):
            pltpu.sync_copy(zbuf, acc_sh.at[pl.ds(s * span + k * zrows, zrows)])
        plsc.subcore_barrier()

        base = (c * NS + s) * EW

        def step(i, cc):
            pltpu.sync_copy(src_hbm.at[pl.ds(base + i * CH, CH)], idx_s)
            pltpu.sync_copy(dst_hbm.at[pl.ds(base + i * CH, CH)], idx_d)
            pltpu.async_copy(u_hbm.at[idx_s], rows_v, sem).wait()
            pltpu.sync_copy(rows_v, acc_sh.at[idx_d], add=True)
            return cc

        lax.fori_loop(0, n_ch, step, 0)
        plsc.subcore_barrier()
        pltpu.sync_copy(acc_sh.at[pl.ds(s * span, span)],
                        out_hbm.at[pl.ds(c * NPAD + s * span, span)])

    return scatter_kernel


def _make_decode_kernel(N, D, LP):
    CH = 128
    EW = LP // NW
    n_ch = EW // CH

    @functools.partial(
        pl.kernel,
        out_type=jax.ShapeDtypeStruct((LP,), jnp.float32),
        mesh=_MESH,
        compiler_params=_SC_PARAMS,
        scratch_types=dict(
            idx_s=pltpu.VMEM((CH,), jnp.int32),
            idx_d=pltpu.VMEM((CH,), jnp.int32),
            zs=pltpu.VMEM((CH, D), jnp.float32),
            zd=pltpu.VMEM((CH, D), jnp.float32),
            outv=pltpu.VMEM((CH,), jnp.float32),
            sem_a=pltpu.SemaphoreType.DMA,
            sem_b=pltpu.SemaphoreType.DMA,
        ),
    )
    def decode_kernel(z_hbm, ls_hbm, ld_hbm, out_hbm,
                      idx_s, idx_d, zs, zd, outv, sem_a, sem_b):
        c = lax.axis_index("c")
        s = lax.axis_index("s")
        base = (c * NS + s) * EW
        lane = lax.iota(jnp.int32, LANES)

        def step(i, cc):
            off = base + i * CH
            pltpu.sync_copy(ls_hbm.at[pl.ds(off, CH)], idx_s)
            pltpu.sync_copy(ld_hbm.at[pl.ds(off, CH)], idx_d)
            ca = pltpu.async_copy(z_hbm.at[idx_s], zs, sem_a)
            cb = pltpu.async_copy(z_hbm.at[idx_d], zd, sem_b)
            ca.wait()
            cb.wait()
            for g in range(CH // LANES):
                rows = g * LANES + lane

                def dot_step(j, acc):
                    cols = (j + lane) & (D - 1)
                    a = plsc.load_gather(zs, [rows, cols])
                    b = plsc.load_gather(zd, [rows, cols])
                    return acc + a * b

                acc = lax.fori_loop(0, D, dot_step,
                                    jnp.zeros((LANES,), jnp.float32))
                outv[pl.ds(g * LANES, LANES)] = acc
            pltpu.sync_copy(outv, out_hbm.at[pl.ds(off, CH)])
            return cc

        lax.fori_loop(0, n_ch, step, 0)

    return decode_kernel


def _tc1_body(x_ref, w_ref, d0_ref, d1_ref, u1_ref, dinv_ref):
    deg = d0_ref[...] + d1_ref[...] + 1.0
    dinv = lax.rsqrt(deg)
    dinv_ref[...] = dinv
    u1_ref[...] = jnp.dot(x_ref[...], w_ref[...],
                          preferred_element_type=jnp.float32) * dinv


def _tc2_body(a0_ref, a1_ref, u1_ref, dinv_ref, b1_ref, w2_ref, u2_ref):
    dinv = dinv_ref[...]
    h = jnp.maximum((a0_ref[...] + a1_ref[...] + u1_ref[...]) * dinv
                    + b1_ref[...], 0.0)
    u2_ref[...] = jnp.dot(h, w2_ref[...],
                          preferred_element_type=jnp.float32) * dinv


def _tc3_body(a0_ref, a1_ref, u2_ref, dinv_ref, b2_ref, z_ref):
    z_ref[...] = ((a0_ref[...] + a1_ref[...] + u2_ref[...]) * dinv_ref[...]
                  + b2_ref[...])


def kernel(x, edge_index, edge_label_index, W1, b1, W2, b2):
    N, Din = x.shape
    Dh = W1.shape[1]
    Do = W2.shape[1]
    E = edge_index.shape[1]
    L = edge_label_index.shape[1]
    NP = ((N + (NS * 16) - 1) // (NS * 16)) * (NS * 16)
    LP = ((L + (NW * 128) - 1) // (NW * 128)) * (NW * 128)

    src = edge_index[0]
    dst = edge_index[1]
    pad = jnp.zeros((LP - L,), jnp.int32)
    ls = jnp.concatenate([edge_label_index[0], pad])
    ld = jnp.concatenate([edge_label_index[1], pad])

    deg_p = _make_deg_kernel(E, NP)(dst)
    d0 = lax.slice(deg_p, (0,), (N,))[:, None]
    d1 = lax.slice(deg_p, (NP,), (NP + N,))[:, None]

    R = 1000
    grid = (N // R,)
    u1, dinv = pl.pallas_call(
        _tc1_body,
        grid=grid,
        in_specs=[
            pl.BlockSpec((R, Din), lambda i: (i, 0)),
            pl.BlockSpec((Din, Dh), lambda i: (0, 0)),
            pl.BlockSpec((R, 1), lambda i: (i, 0)),
            pl.BlockSpec((R, 1), lambda i: (i, 0)),
        ],
        out_specs=[
            pl.BlockSpec((R, Dh), lambda i: (i, 0)),
            pl.BlockSpec((R, 1), lambda i: (i, 0)),
        ],
        out_shape=[
            jax.ShapeDtypeStruct((N, Dh), jnp.float32),
            jax.ShapeDtypeStruct((N, 1), jnp.float32),
        ],
    )(x, W1, d0, d1)

    acc1 = _make_scatter_kernel(NP, Dh, E)(u1, src, dst)
    a10 = lax.slice(acc1, (0, 0), (N, Dh))
    a11 = lax.slice(acc1, (NP, 0), (NP + N, Dh))

    u2 = pl.pallas_call(
        _tc2_body,
        grid=grid,
        in_specs=[
            pl.BlockSpec((R, Dh), lambda i: (i, 0)),
            pl.BlockSpec((R, Dh), lambda i: (i, 0)),
            pl.BlockSpec((R, Dh), lambda i: (i, 0)),
            pl.BlockSpec((R, 1), lambda i: (i, 0)),
            pl.BlockSpec((1, Dh), lambda i: (0, 0)),
            pl.BlockSpec((Dh, Do), lambda i: (0, 0)),
        ],
        out_specs=pl.BlockSpec((R, Do), lambda i: (i, 0)),
        out_shape=jax.ShapeDtypeStruct((N, Do), jnp.float32),
    )(a10, a11, u1, dinv, b1[None, :], W2)

    acc2 = _make_scatter_kernel(NP, Do, E)(u2, src, dst)
    a20 = lax.slice(acc2, (0, 0), (N, Do))
    a21 = lax.slice(acc2, (NP, 0), (NP + N, Do))

    z = pl.pallas_call(
        _tc3_body,
        grid=grid,
        in_specs=[
            pl.BlockSpec((R, Do), lambda i: (i, 0)),
            pl.BlockSpec((R, Do), lambda i: (i, 0)),
            pl.BlockSpec((R, Do), lambda i: (i, 0)),
            pl.BlockSpec((R, 1), lambda i: (i, 0)),
            pl.BlockSpec((1, Do), lambda i: (0, 0)),
        ],
        out_specs=pl.BlockSpec((R, Do), lambda i: (i, 0)),
        out_shape=jax.ShapeDtypeStruct((N, Do), jnp.float32),
    )(a20, a21, u2, dinv, b2[None, :])

    logits_p = _make_decode_kernel(N, Do, LP)(z, ls, ld)
    return lax.slice(logits_p, (0,), (L,))
